# baseline (speedup 1.0000x reference)
import jax
import jax.numpy as jnp
from jax.experimental import pallas as pl
from jax.experimental.pallas import tpu as pltpu

_BLOCK = 2048


def _router_body(x_ref, w_ref, probs_ref, logits_ref):
    x = x_ref[...]
    w = w_ref[...]
    logits = jnp.dot(x, w, preferred_element_type=jnp.float32)
    m = jnp.max(logits, axis=-1, keepdims=True)
    e = jnp.exp(logits - m)
    probs = e / jnp.sum(e, axis=-1, keepdims=True)
    probs_ref[...] = probs
    logits_ref[...] = logits


def kernel(inputs, num_experts, w):
    n_tokens, d = inputs.shape
    n_exp = w.shape[1]
    grid = (n_tokens // _BLOCK,)
    probs, logits = pl.pallas_call(
        _router_body,
        grid=grid,
        in_specs=[
            pl.BlockSpec((_BLOCK, d), lambda i: (i, 0)),
            pl.BlockSpec((d, n_exp), lambda i: (0, 0)),
        ],
        out_specs=[
            pl.BlockSpec((_BLOCK, n_exp), lambda i: (i, 0)),
            pl.BlockSpec((_BLOCK, n_exp), lambda i: (i, 0)),
        ],
        out_shape=[
            jax.ShapeDtypeStruct((n_tokens, n_exp), jnp.float32),
            jax.ShapeDtypeStruct((n_tokens, n_exp), jnp.float32),
        ],
        compiler_params=pltpu.CompilerParams(
            dimension_semantics=("parallel",),
        ),
    )(inputs, w)
    return (probs, logits, 0)


# speedup vs baseline: 1.0206x; 1.0206x over previous
import jax
import jax.numpy as jnp
from jax import lax
from jax.experimental import pallas as pl
from jax.experimental.pallas import tpu as pltpu

_BLOCK = 2048


def _router_body(x_ref, w_ref, probs_ref, logits_ref):
    x = x_ref[...]
    w = w_ref[...]
    logits = jnp.dot(x, w, preferred_element_type=jnp.float32)
    m = jnp.max(logits, axis=-1, keepdims=True)
    e = jnp.exp(logits - m)
    probs = e / jnp.sum(e, axis=-1, keepdims=True)
    probs_ref[...] = probs
    logits_ref[...] = logits


_NSTREAM = 4


def kernel(inputs, num_experts, w):
    n_tokens, d = inputs.shape
    n_exp = w.shape[1]
    n_blocks = n_tokens // _BLOCK
    per = n_blocks // _NSTREAM

    def perm(i):
        return lax.rem(i, _NSTREAM) * per + lax.div(i, _NSTREAM)

    probs, logits = pl.pallas_call(
        _router_body,
        grid=(n_blocks,),
        in_specs=[
            pl.BlockSpec((_BLOCK, d), lambda i: (perm(i), 0)),
            pl.BlockSpec((d, n_exp), lambda i: (0, 0)),
        ],
        out_specs=[
            pl.BlockSpec((_BLOCK, n_exp), lambda i: (perm(i), 0)),
            pl.BlockSpec((_BLOCK, n_exp), lambda i: (perm(i), 0)),
        ],
        out_shape=[
            jax.ShapeDtypeStruct((n_tokens, n_exp), jnp.float32),
            jax.ShapeDtypeStruct((n_tokens, n_exp), jnp.float32),
        ],
        compiler_params=pltpu.CompilerParams(
            dimension_semantics=("parallel",),
        ),
    )(inputs, w)
    return (probs, logits, 0)
